# baseline (speedup 1.0000x reference)
import functools
import math

import jax
import jax.numpy as jnp
from jax import lax
from jax.experimental import pallas as pl
from jax.experimental.pallas import tpu as pltpu
from jax.experimental.pallas import tpu_sc as plsc

NUM_SAMPLED = 1024
NUM_CLASSES = 100000
LAMB = 0.1
BATCH = 4096
DIM = 128
N_CODES = 4

_NW = 32
_TB = BATCH // _NW
_SB = NUM_SAMPLED // _NW
_LOGNC1 = math.log(NUM_CLASSES + 1.0)


def _sc_gather(table, bias, labels, sampled):
    mesh = plsc.VectorSubcoreMesh(core_axis_name="c", subcore_axis_name="s")

    @functools.partial(
        pl.kernel,
        out_type=(
            jax.ShapeDtypeStruct((BATCH, DIM), jnp.float32),
            jax.ShapeDtypeStruct((BATCH,), jnp.float32),
            jax.ShapeDtypeStruct((NUM_SAMPLED, DIM), jnp.float32),
            jax.ShapeDtypeStruct((NUM_SAMPLED,), jnp.float32),
        ),
        mesh=mesh,
        scratch_types=[
            pltpu.VMEM((_TB,), jnp.int32),
            pltpu.VMEM((_SB,), jnp.int32),
            pltpu.VMEM((_TB, DIM), jnp.float32),
            pltpu.VMEM((_SB, DIM), jnp.float32),
            pltpu.VMEM((_TB,), jnp.float32),
            pltpu.VMEM((_SB,), jnp.float32),
        ] + [pltpu.SemaphoreType.DMA] * 8,
    )
    def k(table_h, bias_h, labels_h, sampled_h,
          tw_h, tb_h, sw_h, sb_h,
          lidx, sidx, twv, swv, tbv, sbv,
          g1, g2, g3, g4, w1, w2, w3, w4):
        wid = lax.axis_index("s") * 2 + lax.axis_index("c")
        tbase = wid * _TB
        sbase = wid * _SB
        pltpu.sync_copy(labels_h.at[pl.ds(tbase, _TB)], lidx)
        pltpu.sync_copy(sampled_h.at[pl.ds(sbase, _SB)], sidx)
        c1 = pltpu.async_copy(table_h.at[lidx], twv, g1)
        c2 = pltpu.async_copy(table_h.at[sidx], swv, g2)
        c3 = pltpu.async_copy(bias_h.at[lidx], tbv, g3)
        c4 = pltpu.async_copy(bias_h.at[sidx], sbv, g4)
        c1.wait()
        o1 = pltpu.async_copy(twv, tw_h.at[pl.ds(tbase, _TB)], w1)
        c2.wait()
        o2 = pltpu.async_copy(swv, sw_h.at[pl.ds(sbase, _SB)], w2)
        c3.wait()
        o3 = pltpu.async_copy(tbv, tb_h.at[pl.ds(tbase, _TB)], w3)
        c4.wait()
        o4 = pltpu.async_copy(sbv, sb_h.at[pl.ds(sbase, _SB)], w4)
        o1.wait()
        o2.wait()
        o3.wait()
        o4.wait()

    return k(table, bias, labels, sampled)


_BB = 512


def _tc_body(xin_ref, lab4_ref, tw_ref, tb_ref, sw_ref, sb_ref, nlq_ref,
             samp_ref, out_ref):
    i = pl.program_id(0)
    x = xin_ref[0]
    s_log = lax.dot_general(x, sw_ref[...], (((1,), (1,)), ((), ())),
                            preferred_element_type=jnp.float32)
    s_log = s_log + (sb_ref[...] + nlq_ref[...])
    lab = lab4_ref[:, 3:4]
    hit = lab == samp_ref[...]
    s_log = jnp.where(hit, s_log - 1e9, s_log)
    labf = lab.astype(jnp.float32)
    tp = (jnp.log(labf + 2.0) - jnp.log(labf + 1.0)) / _LOGNC1
    tq = 1.0 - jnp.exp(NUM_SAMPLED * jnp.log(1.0 - tp))
    t_log = (jnp.sum(x * tw_ref[...], axis=1, keepdims=True)
             + tb_ref[...] - jnp.log(tq + 1e-20))
    se = jnp.sum(jnp.exp(s_log), axis=1, keepdims=True) + jnp.exp(t_log)
    loss = jnp.log(se) - t_log

    @pl.when(i == 0)
    def _():
        out_ref[0, 0] = 0.0

    out_ref[0, 0] += jnp.sum(loss)

    @pl.when(i == BATCH // _BB - 1)
    def _():
        out_ref[0, 0] *= jnp.float32(LAMB / BATCH)


def _tc_loss(inputs_in, labels4, tw, tb_col, sw, sb_row, nlq_row, samp_row):
    grid = BATCH // _BB
    return pl.pallas_call(
        _tc_body,
        grid=(grid,),
        in_specs=[
            pl.BlockSpec((1, _BB, DIM), lambda i: (2, i, 0)),
            pl.BlockSpec((_BB, N_CODES), lambda i: (i, 0)),
            pl.BlockSpec((_BB, DIM), lambda i: (i, 0)),
            pl.BlockSpec((_BB, 1), lambda i: (i, 0)),
            pl.BlockSpec((NUM_SAMPLED, DIM), lambda i: (0, 0)),
            pl.BlockSpec((1, NUM_SAMPLED), lambda i: (0, 0)),
            pl.BlockSpec((1, NUM_SAMPLED), lambda i: (0, 0)),
            pl.BlockSpec((1, NUM_SAMPLED), lambda i: (0, 0)),
        ],
        out_specs=pl.BlockSpec(memory_space=pltpu.SMEM),
        out_shape=jax.ShapeDtypeStruct((1, 1), jnp.float32),
    )(inputs_in, labels4, tw, tb_col, sw, sb_row, nlq_row, samp_row)


def _sampled_ids():
    key = jax.random.fold_in(jax.random.key(42), 5)
    u = jax.random.uniform(key, (NUM_SAMPLED,))
    s = jnp.floor(jnp.exp(u * jnp.log(NUM_CLASSES + 1.0))) - 1.0
    return jnp.clip(s, 0, NUM_CLASSES - 1).astype(jnp.int32)


def kernel(inputs_in, labels_in, kernel, bias):
    labels4 = labels_in.astype(jnp.int32)
    labels = labels4[:, 3]
    sampled = _sampled_ids()
    samp_f = sampled.astype(jnp.float32)
    sp = (jnp.log(samp_f + 2.0) - jnp.log(samp_f + 1.0)) / _LOGNC1
    sq = 1.0 - jnp.exp(NUM_SAMPLED * jnp.log(1.0 - sp))
    nlq_row = (-jnp.log(sq + 1e-20)).reshape(1, NUM_SAMPLED)
    samp_row = sampled.reshape(1, NUM_SAMPLED)

    tw, tb, sw, sb = _sc_gather(kernel, bias, labels, sampled)
    out = _tc_loss(
        jnp.transpose(inputs_in.astype(jnp.float32), (1, 0, 2)),
        labels4,
        tw,
        tb.reshape(BATCH, 1),
        sw,
        sb.reshape(1, NUM_SAMPLED),
        nlq_row,
        samp_row,
    )
    return out[0, 0]


# speedup vs baseline: 1.0885x; 1.0885x over previous
import functools
import math

import jax
import jax.numpy as jnp
from jax import lax
from jax.experimental import pallas as pl
from jax.experimental.pallas import tpu as pltpu
from jax.experimental.pallas import tpu_sc as plsc

NUM_SAMPLED = 1024
NUM_CLASSES = 100000
LAMB = 0.1
BATCH = 4096
DIM = 128
N_CODES = 4

_NW = 32
_TB = BATCH // _NW
_SB = NUM_SAMPLED // _NW
_LOGNC1 = math.log(NUM_CLASSES + 1.0)


def _sc_gather(table, bias, labels, sampled):
    mesh = plsc.VectorSubcoreMesh(core_axis_name="c", subcore_axis_name="s")

    @functools.partial(
        pl.kernel,
        out_type=(
            jax.ShapeDtypeStruct((BATCH, DIM), jnp.float32),
            jax.ShapeDtypeStruct((BATCH,), jnp.float32),
            jax.ShapeDtypeStruct((NUM_SAMPLED, DIM), jnp.float32),
            jax.ShapeDtypeStruct((NUM_SAMPLED,), jnp.float32),
        ),
        mesh=mesh,
        scratch_types=[
            pltpu.VMEM((_TB,), jnp.int32),
            pltpu.VMEM((_SB,), jnp.int32),
            pltpu.VMEM((_TB, DIM), jnp.float32),
            pltpu.VMEM((_SB, DIM), jnp.float32),
            pltpu.VMEM((_TB,), jnp.float32),
            pltpu.VMEM((_SB,), jnp.float32),
        ] + [pltpu.SemaphoreType.DMA] * 8,
    )
    def k(table_h, bias_h, labels_h, sampled_h,
          tw_h, tb_h, sw_h, sb_h,
          lidx, sidx, twv, swv, tbv, sbv,
          g1, g2, g3, g4, w1, w2, w3, w4):
        wid = lax.axis_index("s") * 2 + lax.axis_index("c")
        tbase = wid * _TB
        sbase = wid * _SB
        pltpu.sync_copy(labels_h.at[pl.ds(tbase, _TB)], lidx)
        pltpu.sync_copy(sampled_h.at[pl.ds(sbase, _SB)], sidx)
        c1 = pltpu.async_copy(table_h.at[lidx], twv, g1)
        c2 = pltpu.async_copy(table_h.at[sidx], swv, g2)
        c3 = pltpu.async_copy(bias_h.at[lidx], tbv, g3)
        c4 = pltpu.async_copy(bias_h.at[sidx], sbv, g4)
        c1.wait()
        o1 = pltpu.async_copy(twv, tw_h.at[pl.ds(tbase, _TB)], w1)
        c2.wait()
        o2 = pltpu.async_copy(swv, sw_h.at[pl.ds(sbase, _SB)], w2)
        c3.wait()
        o3 = pltpu.async_copy(tbv, tb_h.at[pl.ds(tbase, _TB)], w3)
        c4.wait()
        o4 = pltpu.async_copy(sbv, sb_h.at[pl.ds(sbase, _SB)], w4)
        o1.wait()
        o2.wait()
        o3.wait()
        o4.wait()

    return k(table, bias, labels, sampled)


_BB = 512


def _extract_body(xin_ref, out_ref):
    out_ref[...] = xin_ref[:, 2, :]


def _tc_extract(inputs_in):
    grid = BATCH // _BB
    return pl.pallas_call(
        _extract_body,
        grid=(grid,),
        in_specs=[pl.BlockSpec((_BB, N_CODES, DIM), lambda i: (i, 0, 0))],
        out_specs=pl.BlockSpec((_BB, DIM), lambda i: (i, 0)),
        out_shape=jax.ShapeDtypeStruct((BATCH, DIM), jnp.float32),
    )(inputs_in)


def _tc_body(x_ref, lab4_ref, tw_ref, tb_ref, sw_ref, sb_ref, nlq_ref,
             samp_ref, ones_ref, out_ref):
    i = pl.program_id(0)
    x = x_ref[...]
    s_log = lax.dot_general(x, sw_ref[...], (((1,), (1,)), ((), ())),
                            preferred_element_type=jnp.float32)
    s_log = s_log + (sb_ref[...] + nlq_ref[...])
    lab = lab4_ref[:, 3:4]
    hit = lab == samp_ref[...]
    s_log = jnp.where(hit, s_log - 1e9, s_log)
    es = jnp.exp(s_log)
    se_s = lax.dot_general(es, ones_ref[...], (((1,), (0,)), ((), ())),
                           preferred_element_type=jnp.float32)[:, :1]
    t_dot = lax.dot_general(x * tw_ref[...], ones_ref[:DIM, :],
                            (((1,), (0,)), ((), ())),
                            preferred_element_type=jnp.float32)[:, :1]
    labf = lab.astype(jnp.float32)
    tp = (jnp.log(labf + 2.0) - jnp.log(labf + 1.0)) / _LOGNC1
    tq = 1.0 - jnp.exp(NUM_SAMPLED * jnp.log(1.0 - tp))
    t_log = t_dot + tb_ref[...] - jnp.log(tq + 1e-20)
    se = se_s + jnp.exp(t_log)
    loss = jnp.log(se) - t_log

    @pl.when(i == 0)
    def _():
        out_ref[0, 0] = 0.0

    out_ref[0, 0] += jnp.sum(loss)

    @pl.when(i == BATCH // _BB - 1)
    def _():
        out_ref[0, 0] *= jnp.float32(LAMB / BATCH)


def _tc_loss(x2, labels4, tw, tb_col, sw, sb_row, nlq_row, samp_row, ones_m):
    grid = BATCH // _BB
    return pl.pallas_call(
        _tc_body,
        grid=(grid,),
        in_specs=[
            pl.BlockSpec((_BB, DIM), lambda i: (i, 0)),
            pl.BlockSpec((_BB, N_CODES), lambda i: (i, 0)),
            pl.BlockSpec((_BB, DIM), lambda i: (i, 0)),
            pl.BlockSpec((_BB, 1), lambda i: (i, 0)),
            pl.BlockSpec((NUM_SAMPLED, DIM), lambda i: (0, 0)),
            pl.BlockSpec((1, NUM_SAMPLED), lambda i: (0, 0)),
            pl.BlockSpec((1, NUM_SAMPLED), lambda i: (0, 0)),
            pl.BlockSpec((1, NUM_SAMPLED), lambda i: (0, 0)),
            pl.BlockSpec((NUM_SAMPLED, DIM), lambda i: (0, 0)),
        ],
        out_specs=pl.BlockSpec(memory_space=pltpu.SMEM),
        out_shape=jax.ShapeDtypeStruct((1, 1), jnp.float32),
    )(x2, labels4, tw, tb_col, sw, sb_row, nlq_row, samp_row, ones_m)


def _sampled_ids():
    key = jax.random.fold_in(jax.random.key(42), 5)
    u = jax.random.uniform(key, (NUM_SAMPLED,))
    s = jnp.floor(jnp.exp(u * jnp.log(NUM_CLASSES + 1.0))) - 1.0
    return jnp.clip(s, 0, NUM_CLASSES - 1).astype(jnp.int32)


def kernel(inputs_in, labels_in, kernel, bias):
    labels4 = labels_in.astype(jnp.int32)
    labels = labels4[:, 3]
    sampled = _sampled_ids()
    samp_f = sampled.astype(jnp.float32)
    sp = (jnp.log(samp_f + 2.0) - jnp.log(samp_f + 1.0)) / _LOGNC1
    sq = 1.0 - jnp.exp(NUM_SAMPLED * jnp.log(1.0 - sp))
    nlq_row = (-jnp.log(sq + 1e-20)).reshape(1, NUM_SAMPLED)
    samp_row = sampled.reshape(1, NUM_SAMPLED)
    ones_m = jnp.ones((NUM_SAMPLED, DIM), jnp.float32)

    x2 = _tc_extract(inputs_in.astype(jnp.float32))
    tw, tb, sw, sb = _sc_gather(kernel, bias, labels, sampled)
    out = _tc_loss(x2, labels4, tw, tb.reshape(BATCH, 1), sw,
                   sb.reshape(1, NUM_SAMPLED), nlq_row, samp_row, ones_m)
    return out[0, 0]


# speedup vs baseline: 1.1870x; 1.0905x over previous
import functools
import math

import jax
import jax.numpy as jnp
import numpy as np
from jax import lax
from jax.experimental import pallas as pl
from jax.experimental.pallas import tpu as pltpu
from jax.experimental.pallas import tpu_sc as plsc

NUM_SAMPLED = 1024
NUM_CLASSES = 100000
LAMB = 0.1
BATCH = 4096
DIM = 128
N_CODES = 4

_NW = 32
_TB = BATCH // _NW
_SB = NUM_SAMPLED // _NW
_LOGNC1 = math.log(NUM_CLASSES + 1.0)


def _sc_gather(table, bias, labels, sampled):
    mesh = plsc.VectorSubcoreMesh(core_axis_name="c", subcore_axis_name="s")

    @functools.partial(
        pl.kernel,
        out_type=(
            jax.ShapeDtypeStruct((BATCH, DIM), jnp.float32),
            jax.ShapeDtypeStruct((BATCH,), jnp.float32),
            jax.ShapeDtypeStruct((NUM_SAMPLED, DIM), jnp.float32),
            jax.ShapeDtypeStruct((NUM_SAMPLED,), jnp.float32),
        ),
        mesh=mesh,
        scratch_types=[
            pltpu.VMEM((_TB,), jnp.int32),
            pltpu.VMEM((_SB,), jnp.int32),
            pltpu.VMEM((_TB, DIM), jnp.float32),
            pltpu.VMEM((_SB, DIM), jnp.float32),
            pltpu.VMEM((_TB,), jnp.float32),
            pltpu.VMEM((_SB,), jnp.float32),
        ] + [pltpu.SemaphoreType.DMA] * 8,
    )
    def k(table_h, bias_h, labels_h, sampled_h,
          tw_h, tb_h, sw_h, sb_h,
          lidx, sidx, twv, swv, tbv, sbv,
          g1, g2, g3, g4, w1, w2, w3, w4):
        wid = lax.axis_index("s") * 2 + lax.axis_index("c")
        tbase = wid * _TB
        sbase = wid * _SB
        pltpu.sync_copy(labels_h.at[pl.ds(tbase, _TB)], lidx)
        pltpu.sync_copy(sampled_h.at[pl.ds(sbase, _SB)], sidx)
        c1 = pltpu.async_copy(table_h.at[lidx], twv, g1)
        c2 = pltpu.async_copy(table_h.at[sidx], swv, g2)
        c3 = pltpu.async_copy(bias_h.at[lidx], tbv, g3)
        c4 = pltpu.async_copy(bias_h.at[sidx], sbv, g4)
        c1.wait()
        o1 = pltpu.async_copy(twv, tw_h.at[pl.ds(tbase, _TB)], w1)
        c2.wait()
        o2 = pltpu.async_copy(swv, sw_h.at[pl.ds(sbase, _SB)], w2)
        c3.wait()
        o3 = pltpu.async_copy(tbv, tb_h.at[pl.ds(tbase, _TB)], w3)
        c4.wait()
        o4 = pltpu.async_copy(sbv, sb_h.at[pl.ds(sbase, _SB)], w4)
        o1.wait()
        o2.wait()
        o3.wait()
        o4.wait()

    return k(table, bias, labels, sampled)


_BB = 512


def _tc_body(xin_ref, lab4_ref, tw_ref, tb_ref, sw_ref, sb_ref, nlq_ref,
             samp_ref, out_ref):
    i = pl.program_id(0)
    x = xin_ref[:, 2, :]
    ones_s = jnp.ones((NUM_SAMPLED, DIM), jnp.float32)
    ones_t = jnp.ones((DIM, DIM), jnp.float32)
    s_log = lax.dot_general(x, sw_ref[...], (((1,), (1,)), ((), ())),
                            preferred_element_type=jnp.float32)
    s_log = s_log + (sb_ref[...] + nlq_ref[...])
    lab = lab4_ref[:, 3:4]
    hit = lab == samp_ref[...]
    s_log = jnp.where(hit, s_log - 1e9, s_log)
    es = jnp.exp(s_log)
    se_s = lax.dot_general(es, ones_s, (((1,), (0,)), ((), ())),
                           preferred_element_type=jnp.float32)[:, :1]
    t_dot = lax.dot_general(x * tw_ref[...], ones_t,
                            (((1,), (0,)), ((), ())),
                            preferred_element_type=jnp.float32)[:, :1]
    labf = lab.astype(jnp.float32)
    tp = (jnp.log(labf + 2.0) - jnp.log(labf + 1.0)) / _LOGNC1
    tq = 1.0 - jnp.exp(NUM_SAMPLED * jnp.log(1.0 - tp))
    t_log = t_dot + tb_ref[...] - jnp.log(tq + 1e-20)
    se = se_s + jnp.exp(t_log)
    loss = jnp.log(se) - t_log

    @pl.when(i == 0)
    def _():
        out_ref[0, 0] = 0.0

    out_ref[0, 0] += jnp.sum(loss)

    @pl.when(i == BATCH // _BB - 1)
    def _():
        out_ref[0, 0] *= jnp.float32(LAMB / BATCH)


def _tc_loss(inputs_in, labels4, tw, tb_col, sw, sb_row):
    grid = BATCH // _BB
    return pl.pallas_call(
        _tc_body,
        grid=(grid,),
        in_specs=[
            pl.BlockSpec((_BB, N_CODES, DIM), lambda i: (i, 0, 0)),
            pl.BlockSpec((_BB, N_CODES), lambda i: (i, 0)),
            pl.BlockSpec((_BB, DIM), lambda i: (i, 0)),
            pl.BlockSpec((_BB, 1), lambda i: (i, 0)),
            pl.BlockSpec((NUM_SAMPLED, DIM), lambda i: (0, 0)),
            pl.BlockSpec((1, NUM_SAMPLED), lambda i: (0, 0)),
            pl.BlockSpec((1, NUM_SAMPLED), lambda i: (0, 0)),
            pl.BlockSpec((1, NUM_SAMPLED), lambda i: (0, 0)),
        ],
        out_specs=pl.BlockSpec(memory_space=pltpu.SMEM),
        out_shape=jax.ShapeDtypeStruct((1, 1), jnp.float32),
    )(inputs_in, labels4, tw, tb_col, sw, sb_row, _NLQ_ROW, _SAMP_ROW)


def _sampled_consts():
    key = jax.random.fold_in(jax.random.key(42), 5)
    u = jax.random.uniform(key, (NUM_SAMPLED,))
    s = jnp.floor(jnp.exp(u * jnp.log(NUM_CLASSES + 1.0))) - 1.0
    sampled = jnp.clip(s, 0, NUM_CLASSES - 1).astype(jnp.int32)
    samp_f = sampled.astype(jnp.float32)
    sp = (jnp.log(samp_f + 2.0) - jnp.log(samp_f + 1.0)) / _LOGNC1
    sq = 1.0 - jnp.exp(NUM_SAMPLED * jnp.log(1.0 - sp))
    nlq = -jnp.log(sq + 1e-20)
    return (np.asarray(sampled), np.asarray(nlq, dtype=np.float32))


_SAMPLED_IDS, _NLQ = _sampled_consts()
_NLQ_ROW = _NLQ.reshape(1, NUM_SAMPLED)
_SAMP_ROW = _SAMPLED_IDS.reshape(1, NUM_SAMPLED)


def kernel(inputs_in, labels_in, kernel, bias):
    labels4 = labels_in.astype(jnp.int32)
    labels = labels4[:, 3]
    tw, tb, sw, sb = _sc_gather(kernel, bias, labels,
                                jnp.asarray(_SAMPLED_IDS))
    out = _tc_loss(inputs_in.astype(jnp.float32), labels4, tw,
                   tb.reshape(BATCH, 1), sw, sb.reshape(1, NUM_SAMPLED))
    return out[0, 0]


# speedup vs baseline: 1.2106x; 1.0199x over previous
import functools
import math

import jax
import jax.numpy as jnp
import numpy as np
from jax import lax
from jax.experimental import pallas as pl
from jax.experimental.pallas import tpu as pltpu
from jax.experimental.pallas import tpu_sc as plsc

NUM_SAMPLED = 1024
NUM_CLASSES = 100000
LAMB = 0.1
BATCH = 4096
DIM = 128
N_CODES = 4

_NW = 32
_TB = BATCH // _NW
_SB = NUM_SAMPLED // _NW
_LOGNC1 = math.log(NUM_CLASSES + 1.0)


def _sc_gather(table, bias, labels, sampled):
    mesh = plsc.VectorSubcoreMesh(core_axis_name="c", subcore_axis_name="s")

    @functools.partial(
        pl.kernel,
        out_type=(
            jax.ShapeDtypeStruct((BATCH, DIM), jnp.float32),
            jax.ShapeDtypeStruct((BATCH,), jnp.float32),
            jax.ShapeDtypeStruct((NUM_SAMPLED, DIM), jnp.float32),
            jax.ShapeDtypeStruct((NUM_SAMPLED,), jnp.float32),
        ),
        mesh=mesh,
        scratch_types=[
            pltpu.VMEM((_TB,), jnp.int32),
            pltpu.VMEM((_SB,), jnp.int32),
            pltpu.VMEM((_TB, DIM), jnp.float32),
            pltpu.VMEM((_SB, DIM), jnp.float32),
            pltpu.VMEM((_TB,), jnp.float32),
            pltpu.VMEM((_SB,), jnp.float32),
        ] + [pltpu.SemaphoreType.DMA] * 8,
    )
    def k(table_h, bias_h, labels_h, sampled_h,
          tw_h, tb_h, sw_h, sb_h,
          lidx, sidx, twv, swv, tbv, sbv,
          g1, g2, g3, g4, w1, w2, w3, w4):
        wid = lax.axis_index("s") * 2 + lax.axis_index("c")
        tbase = wid * _TB
        sbase = wid * _SB
        pltpu.sync_copy(labels_h.at[pl.ds(tbase, _TB)], lidx)
        pltpu.sync_copy(sampled_h.at[pl.ds(sbase, _SB)], sidx)
        c1 = pltpu.async_copy(table_h.at[lidx], twv, g1)
        c2 = pltpu.async_copy(table_h.at[sidx], swv, g2)
        c3 = pltpu.async_copy(bias_h.at[lidx], tbv, g3)
        c4 = pltpu.async_copy(bias_h.at[sidx], sbv, g4)
        c1.wait()
        o1 = pltpu.async_copy(twv, tw_h.at[pl.ds(tbase, _TB)], w1)
        c2.wait()
        o2 = pltpu.async_copy(swv, sw_h.at[pl.ds(sbase, _SB)], w2)
        c3.wait()
        o3 = pltpu.async_copy(tbv, tb_h.at[pl.ds(tbase, _TB)], w3)
        c4.wait()
        o4 = pltpu.async_copy(sbv, sb_h.at[pl.ds(sbase, _SB)], w4)
        o1.wait()
        o2.wait()
        o3.wait()
        o4.wait()

    return k(table, bias, labels, sampled)


_BB = 512


def _tc_body(xin_ref, lab4_ref, tw_ref, tb_ref, sw_ref, sb_ref, nlq_ref,
             samp_ref, out_ref):
    i = pl.program_id(0)
    x = xin_ref[:, 2, :]
    ones_s = jnp.ones((NUM_SAMPLED, DIM), jnp.float32)
    ones_t = jnp.ones((DIM, DIM), jnp.float32)
    s_log = lax.dot_general(x, sw_ref[...], (((1,), (1,)), ((), ())),
                            preferred_element_type=jnp.float32)
    s_log = s_log + (sb_ref[...] + nlq_ref[...])
    lab = lab4_ref[:, 3:4]
    hit = lab == samp_ref[...]
    s_log = jnp.where(hit, s_log - 1e9, s_log)
    es = jnp.exp(s_log)
    se_s = lax.dot_general(es, ones_s, (((1,), (0,)), ((), ())),
                           preferred_element_type=jnp.float32)[:, :1]
    t_dot = lax.dot_general(x * tw_ref[...], ones_t,
                            (((1,), (0,)), ((), ())),
                            preferred_element_type=jnp.float32)[:, :1]
    labf = lab.astype(jnp.float32)
    tp = (jnp.log(labf + 2.0) - jnp.log(labf + 1.0)) / _LOGNC1
    tq = 1.0 - jnp.exp(NUM_SAMPLED * jnp.log(1.0 - tp))
    t_log = t_dot + tb_ref[...] - jnp.log(tq + 1e-20)
    se = se_s + jnp.exp(t_log)
    loss = jnp.log(se) - t_log

    @pl.when(i == 0)
    def _():
        out_ref[0, 0] = 0.0

    out_ref[0, 0] += jnp.sum(loss)

    @pl.when(i == BATCH // _BB - 1)
    def _():
        out_ref[0, 0] *= jnp.float32(LAMB / BATCH)


def _tc_loss(inputs_in, labels4, tw, tb_col, sw, sb_row, nlq_row, samp_row):
    grid = BATCH // _BB
    return pl.pallas_call(
        _tc_body,
        grid=(grid,),
        in_specs=[
            pl.BlockSpec((_BB, N_CODES, DIM), lambda i: (i, 0, 0)),
            pl.BlockSpec((_BB, N_CODES), lambda i: (i, 0)),
            pl.BlockSpec((_BB, DIM), lambda i: (i, 0)),
            pl.BlockSpec((_BB, 1), lambda i: (i, 0)),
            pl.BlockSpec((NUM_SAMPLED, DIM), lambda i: (0, 0)),
            pl.BlockSpec((1, NUM_SAMPLED), lambda i: (0, 0)),
            pl.BlockSpec((1, NUM_SAMPLED), lambda i: (0, 0)),
            pl.BlockSpec((1, NUM_SAMPLED), lambda i: (0, 0)),
        ],
        out_specs=pl.BlockSpec(memory_space=pltpu.SMEM),
        out_shape=jax.ShapeDtypeStruct((1, 1), jnp.float32),
    )(inputs_in, labels4, tw, tb_col, sw, sb_row, nlq_row, samp_row)


def kernel(inputs_in, labels_in, kernel, bias):
    labels4 = labels_in.astype(jnp.int32)
    labels = labels4[:, 3]
    key = jax.random.fold_in(jax.random.key(42), 5)
    u = jax.random.uniform(key, (NUM_SAMPLED,))
    s = jnp.floor(jnp.exp(u * jnp.log(NUM_CLASSES + 1.0))) - 1.0
    sampled = jnp.clip(s, 0, NUM_CLASSES - 1).astype(jnp.int32)
    samp_f = sampled.astype(jnp.float32)
    sp = (jnp.log(samp_f + 2.0) - jnp.log(samp_f + 1.0)) / _LOGNC1
    sq = 1.0 - jnp.exp(NUM_SAMPLED * jnp.log(1.0 - sp))
    nlq_row = (-jnp.log(sq + 1e-20)).reshape(1, NUM_SAMPLED)
    samp_row = sampled.reshape(1, NUM_SAMPLED)

    tw, tb, sw, sb = _sc_gather(kernel, bias, labels, sampled)
    out = _tc_loss(inputs_in.astype(jnp.float32), labels4, tw,
                   tb.reshape(BATCH, 1), sw, sb.reshape(1, NUM_SAMPLED),
                   nlq_row, samp_row)
    return out[0, 0]


# speedup vs baseline: 1.2201x; 1.0079x over previous
import functools
import math

import jax
import jax.numpy as jnp
import numpy as np
from jax import lax
from jax.experimental import pallas as pl
from jax.experimental.pallas import tpu as pltpu
from jax.experimental.pallas import tpu_sc as plsc

NUM_SAMPLED = 1024
NUM_CLASSES = 100000
LAMB = 0.1
BATCH = 4096
DIM = 128
N_CODES = 4

_NW = 32
_TB = BATCH // _NW
_SB = NUM_SAMPLED // _NW
_LOGNC1 = math.log(NUM_CLASSES + 1.0)


def _sc_gather(table, bias, labels, sampled):
    mesh = plsc.VectorSubcoreMesh(core_axis_name="c", subcore_axis_name="s")

    @functools.partial(
        pl.kernel,
        out_type=(
            jax.ShapeDtypeStruct((BATCH, DIM), jnp.float32),
            jax.ShapeDtypeStruct((BATCH,), jnp.float32),
            jax.ShapeDtypeStruct((NUM_SAMPLED, DIM), jnp.float32),
            jax.ShapeDtypeStruct((NUM_SAMPLED,), jnp.float32),
        ),
        mesh=mesh,
        scratch_types=[
            pltpu.VMEM((_TB,), jnp.int32),
            pltpu.VMEM((_SB,), jnp.int32),
            pltpu.VMEM((_TB, DIM), jnp.float32),
            pltpu.VMEM((_SB, DIM), jnp.float32),
            pltpu.VMEM((_TB,), jnp.float32),
            pltpu.VMEM((_SB,), jnp.float32),
        ] + [pltpu.SemaphoreType.DMA] * 8,
    )
    def k(table_h, bias_h, labels_h, sampled_h,
          tw_h, tb_h, sw_h, sb_h,
          lidx, sidx, twv, swv, tbv, sbv,
          g1, g2, g3, g4, w1, w2, w3, w4):
        wid = lax.axis_index("s") * 2 + lax.axis_index("c")
        tbase = wid * _TB
        sbase = wid * _SB
        pltpu.sync_copy(labels_h.at[pl.ds(tbase, _TB)], lidx)
        pltpu.sync_copy(sampled_h.at[pl.ds(sbase, _SB)], sidx)
        c1 = pltpu.async_copy(table_h.at[lidx], twv, g1)
        c2 = pltpu.async_copy(table_h.at[sidx], swv, g2)
        c3 = pltpu.async_copy(bias_h.at[lidx], tbv, g3)
        c4 = pltpu.async_copy(bias_h.at[sidx], sbv, g4)
        c1.wait()
        o1 = pltpu.async_copy(twv, tw_h.at[pl.ds(tbase, _TB)], w1)
        c2.wait()
        o2 = pltpu.async_copy(swv, sw_h.at[pl.ds(sbase, _SB)], w2)
        c3.wait()
        o3 = pltpu.async_copy(tbv, tb_h.at[pl.ds(tbase, _TB)], w3)
        c4.wait()
        o4 = pltpu.async_copy(sbv, sb_h.at[pl.ds(sbase, _SB)], w4)
        o1.wait()
        o2.wait()
        o3.wait()
        o4.wait()

    return k(table, bias, labels, sampled)


_BB = 1024


def _tc_body(xin_ref, lab4_ref, tw_ref, tb_ref, sw_ref, sb_ref, nlq_ref,
             samp_ref, out_ref):
    i = pl.program_id(0)
    x = xin_ref[:, 2, :]
    ones_s = jnp.ones((NUM_SAMPLED, DIM), jnp.float32)
    ones_t = jnp.ones((DIM, DIM), jnp.float32)
    s_log = lax.dot_general(x, sw_ref[...], (((1,), (1,)), ((), ())),
                            preferred_element_type=jnp.float32)
    s_log = s_log + (sb_ref[...] + nlq_ref[...])
    lab = lab4_ref[:, 3:4]
    hit = lab == samp_ref[...]
    s_log = jnp.where(hit, s_log - 1e9, s_log)
    es = jnp.exp(s_log)
    se_s = lax.dot_general(es, ones_s, (((1,), (0,)), ((), ())),
                           preferred_element_type=jnp.float32)[:, :1]
    t_dot = lax.dot_general(x * tw_ref[...], ones_t,
                            (((1,), (0,)), ((), ())),
                            preferred_element_type=jnp.float32)[:, :1]
    labf = lab.astype(jnp.float32)
    tp = (jnp.log(labf + 2.0) - jnp.log(labf + 1.0)) / _LOGNC1
    tq = 1.0 - jnp.exp(NUM_SAMPLED * jnp.log(1.0 - tp))
    t_log = t_dot + tb_ref[...] - jnp.log(tq + 1e-20)
    se = se_s + jnp.exp(t_log)
    loss = jnp.log(se) - t_log

    @pl.when(i == 0)
    def _():
        out_ref[0, 0] = 0.0

    out_ref[0, 0] += jnp.sum(loss)

    @pl.when(i == BATCH // _BB - 1)
    def _():
        out_ref[0, 0] *= jnp.float32(LAMB / BATCH)


def _tc_loss(inputs_in, labels4, tw, tb_col, sw, sb_row, nlq_row, samp_row):
    grid = BATCH // _BB
    return pl.pallas_call(
        _tc_body,
        grid=(grid,),
        in_specs=[
            pl.BlockSpec((_BB, N_CODES, DIM), lambda i: (i, 0, 0)),
            pl.BlockSpec((_BB, N_CODES), lambda i: (i, 0)),
            pl.BlockSpec((_BB, DIM), lambda i: (i, 0)),
            pl.BlockSpec((_BB, 1), lambda i: (i, 0)),
            pl.BlockSpec((NUM_SAMPLED, DIM), lambda i: (0, 0)),
            pl.BlockSpec((1, NUM_SAMPLED), lambda i: (0, 0)),
            pl.BlockSpec((1, NUM_SAMPLED), lambda i: (0, 0)),
            pl.BlockSpec((1, NUM_SAMPLED), lambda i: (0, 0)),
        ],
        out_specs=pl.BlockSpec(memory_space=pltpu.SMEM),
        out_shape=jax.ShapeDtypeStruct((1, 1), jnp.float32),
    )(inputs_in, labels4, tw, tb_col, sw, sb_row, nlq_row, samp_row)


def kernel(inputs_in, labels_in, kernel, bias):
    labels4 = labels_in.astype(jnp.int32)
    labels = labels4[:, 3]
    key = jax.random.fold_in(jax.random.key(42), 5)
    u = jax.random.uniform(key, (NUM_SAMPLED,))
    s = jnp.floor(jnp.exp(u * jnp.log(NUM_CLASSES + 1.0))) - 1.0
    sampled = jnp.clip(s, 0, NUM_CLASSES - 1).astype(jnp.int32)
    samp_f = sampled.astype(jnp.float32)
    sp = (jnp.log(samp_f + 2.0) - jnp.log(samp_f + 1.0)) / _LOGNC1
    sq = 1.0 - jnp.exp(NUM_SAMPLED * jnp.log(1.0 - sp))
    nlq_row = (-jnp.log(sq + 1e-20)).reshape(1, NUM_SAMPLED)
    samp_row = sampled.reshape(1, NUM_SAMPLED)

    tw, tb, sw, sb = _sc_gather(kernel, bias, labels, sampled)
    out = _tc_loss(inputs_in.astype(jnp.float32), labels4, tw,
                   tb.reshape(BATCH, 1), sw, sb.reshape(1, NUM_SAMPLED),
                   nlq_row, samp_row)
    return out[0, 0]
